# baseline (speedup 1.0000x reference)
import jax
import jax.numpy as jnp
from jax.experimental import pallas as pl
from jax.experimental.pallas import tpu as pltpu

_B, _I, _J, _NI, _NO = 32, 1024, 64, 16, 16
_O = _J * _NO
_TOPK = 16
_EPS = 1e-8

_ITM = 64
_G1 = _I // _ITM
_CH = 64
_NCH = _I // _CH
_CHT = 256
_NCHT = _I // _CHT


def _rb(x):
    return x.astype(jnp.bfloat16).astype(jnp.float32)


def _uhat_kernel(d_ref, w_ref, out_ref):
    for i in range(_ITM):
        d = d_ref[:, i, :].astype(jnp.bfloat16)
        r = jnp.dot(d, w_ref[i], preferred_element_type=jnp.float32)
        out_ref[:, i, :] = r.astype(jnp.bfloat16)


def _squash_flat(s):
    s2 = s * s
    sq = None
    for n in range(_NO):
        part = s2[:, n * _J:(n + 1) * _J]
        sq = part if sq is None else sq + part
    fac = (sq / (1.0 + sq)) / jnp.sqrt(sq + _EPS)
    facexp = jnp.concatenate([fac] * _NO, axis=1)
    return s * facexp


def _route_kernel(ut_ref, bias_ref, out_ref, bvec_ref, c_ref):
    bias = bias_ref[...]

    def uh(ci):
        return ut_ref[0, pl.ds(ci * _CH, _CH), :].astype(jnp.float32)

    def s0_body(ci, acc):
        return acc + jnp.sum(uh(ci), axis=0, keepdims=True)
    acc = jax.lax.fori_loop(0, _NCH, s0_body,
                            jnp.zeros((1, _O), jnp.float32))
    v = _squash_flat(acc * (1.0 / _J) + bias)

    def bupdate(vflat, first):
        vb = _rb(vflat)
        def body(ci, carry):
            z = uh(ci) * vb
            d = None
            for n in range(_NO):
                zz = z[:, n * _J:(n + 1) * _J]
                d = zz if d is None else d + zz
            sl = pl.ds(ci * _CH, _CH)
            bvec_ref[sl, :] = d if first else bvec_ref[sl, :] + d
            return carry
        jax.lax.fori_loop(0, _NCH, body, 0)

    def topk_softmax():
        def body(tc, carry):
            sl = pl.ds(tc * _CHT, _CHT)
            t = bvec_ref[sl, :]
            gmax = jnp.max(t, axis=1, keepdims=True)
            kept = jnp.sum(jnp.where(t == gmax, 1.0, 0.0),
                           axis=1, keepdims=True)
            tau = gmax
            w = jnp.where(t == gmax, -jnp.inf, t)
            for _step in range(_TOPK - 1):
                m = jnp.max(w, axis=1, keepdims=True)
                cnt = jnp.sum(jnp.where(w == m, 1.0, 0.0),
                              axis=1, keepdims=True)
                active = kept < _TOPK
                tau = jnp.where(active, m, tau)
                kept = kept + jnp.where(active, cnt, 0.0)
                w = jnp.where(w == m, -jnp.inf, w)
            e = jnp.where(t >= tau, jnp.exp(t - gmax), 0.0)
            c_ref[sl, :] = e / jnp.sum(e, axis=1, keepdims=True)
            return carry
        jax.lax.fori_loop(0, _NCHT, body, 0)

    def sstep():
        def body(ci, acc):
            cc = _rb(c_ref[pl.ds(ci * _CH, _CH), :])
            ccexp = jnp.concatenate([cc] * _NO, axis=1)
            return acc + jnp.sum(ccexp * uh(ci), axis=0, keepdims=True)
        return jax.lax.fori_loop(0, _NCH, body,
                                 jnp.zeros((1, _O), jnp.float32)) + bias

    bupdate(v, True)
    for it in (1, 2):
        topk_softmax()
        v = _squash_flat(sstep())
        if it == 1:
            bupdate(v, False)

    out_ref[0] = v


def kernel(data, W, bias):
    Wt = (W.reshape(_I, _J, _NO, _NI).transpose(0, 3, 2, 1)
           .reshape(_I, _NI, _O).astype(jnp.bfloat16))
    bias_flat = bias.transpose(1, 0).reshape(1, _O)

    ut = pl.pallas_call(
        _uhat_kernel,
        grid=(_G1,),
        in_specs=[
            pl.BlockSpec((_B, _ITM, _NI), lambda g: (0, g, 0)),
            pl.BlockSpec((_ITM, _NI, _O), lambda g: (g, 0, 0)),
        ],
        out_specs=pl.BlockSpec((_B, _ITM, _O), lambda g: (0, g, 0)),
        out_shape=jax.ShapeDtypeStruct((_B, _I, _O), jnp.bfloat16),
    )(data, Wt)

    vout = pl.pallas_call(
        _route_kernel,
        grid=(_B,),
        in_specs=[
            pl.BlockSpec((1, _I, _O), lambda b: (b, 0, 0)),
            pl.BlockSpec((1, _O), lambda b: (0, 0)),
        ],
        out_specs=pl.BlockSpec((1, 1, _O), lambda b: (b, 0, 0)),
        out_shape=jax.ShapeDtypeStruct((_B, 1, _O), jnp.float32),
        scratch_shapes=[
            pltpu.VMEM((_I, _J), jnp.float32),
            pltpu.VMEM((_I, _J), jnp.float32),
        ],
    )(ut, bias_flat)

    return vout.reshape(_B, _NO, _J).transpose(0, 2, 1)
